# baseline (speedup 1.0000x reference)
import functools

import jax
import jax.numpy as jnp
from jax import lax
from jax.experimental import pallas as pl
from jax.experimental.pallas import tpu as pltpu
from jax.experimental.pallas import tpu_sc as plsc

N = 10000
E = 320000
D = 128
DE = 16
H = 64
G = 64
C = 2

NC = 2
NS = 16
NW = NC * NS

NP = 10240
EP = 327680
EPT = EP // NW
CH = 128
NCHUNK = EPT // CH
RPT = NP // NS

_HI = jax.lax.Precision.HIGHEST


def _sc_prep_body(ea_hbm, src_hbm, dst_hbm, sege_out, csrc_out, cdst_out,
                  idx_s, idx_d, rows, ones_v, acc_e, acc_s, acc_d):
    cid = lax.axis_index("c")
    sid = lax.axis_index("s")
    wid = sid * NC + cid

    def zfill(i, _):
        rows[i, :] = jnp.zeros((DE,), jnp.float32)
        return 0
    lax.fori_loop(0, CH, zfill, 0)
    for k in range(RPT // CH):
        off = sid * RPT + k * CH
        pltpu.sync_copy(rows, acc_e.at[pl.ds(off, CH)])
        pltpu.sync_copy(rows, acc_s.at[pl.ds(off, CH)])
        pltpu.sync_copy(rows, acc_d.at[pl.ds(off, CH)])

    def fill(i, _):
        ones_v[i, :] = jnp.ones((DE,), jnp.float32)
        return 0
    lax.fori_loop(0, CH, fill, 0)
    plsc.subcore_barrier()

    def body(c, _):
        base = wid * EPT + c * CH
        pltpu.sync_copy(src_hbm.at[pl.ds(base, CH)], idx_s)
        pltpu.sync_copy(dst_hbm.at[pl.ds(base, CH)], idx_d)
        pltpu.sync_copy(ea_hbm.at[pl.ds(base, CH)], rows)
        pltpu.sync_copy(rows, acc_e.at[idx_s], add=True)
        pltpu.sync_copy(ones_v, acc_s.at[idx_s], add=True)
        pltpu.sync_copy(ones_v, acc_d.at[idx_d], add=True)
        return 0
    lax.fori_loop(0, NCHUNK, body, 0)

    plsc.subcore_barrier()
    pltpu.sync_copy(acc_e.at[pl.ds(sid * RPT, RPT)],
                    sege_out.at[cid, pl.ds(sid * RPT, RPT)])
    pltpu.sync_copy(acc_s.at[pl.ds(sid * RPT, RPT)],
                    csrc_out.at[cid, pl.ds(sid * RPT, RPT)])
    pltpu.sync_copy(acc_d.at[pl.ds(sid * RPT, RPT)],
                    cdst_out.at[cid, pl.ds(sid * RPT, RPT)])


NB = 4
A = 2


def _sc_edge_agg_body(y_hbm, src_hbm, dst3_hbm, z_out, *bufs):
    rows = bufs[0:NB]
    sems_g = bufs[NB:2 * NB]
    sems_s = bufs[2 * NB:3 * NB]
    isa = bufs[3 * NB]
    ida = bufs[3 * NB + 1]
    acc = bufs[3 * NB + 2]
    cid = lax.axis_index("c")
    sid = lax.axis_index("s")
    wid = sid * NC + cid

    pltpu.sync_copy(src_hbm.at[pl.ds(wid * EPT, EPT)], isa)
    pltpu.sync_copy(dst3_hbm.at[pl.ds(wid * NCHUNK, NCHUNK)], ida)

    def zfill(i, _):
        for j in range(H // 16):
            rows[0][i, pl.ds(j * 16, 16)] = jnp.zeros((16,), jnp.float32)
        return 0
    lax.fori_loop(0, CH, zfill, 0)
    for k in range(RPT // CH):
        pltpu.sync_copy(rows[0], acc.at[pl.ds(sid * RPT + k * CH, CH)])
    plsc.subcore_barrier()

    for b in range(A):
        pltpu.async_copy(
            y_hbm.at[isa.at[pl.ds(b * CH, CH)]], rows[b], sems_g[b])

    def outer(i, _):
        for b in range(NB):
            c = i * NB + b
            pltpu.make_async_copy(
                y_hbm.at[isa.at[pl.ds(c * CH, CH)]],
                rows[b], sems_g[b]).wait()
            pltpu.async_copy(
                rows[b], acc.at[ida.at[c, 0]], sems_s[b], add=True)

            bn = (b + A) % NB

            @pl.when(c + A < NCHUNK)
            def _():
                @pl.when(c + A >= NB)
                def _():
                    pltpu.make_async_copy(
                        rows[bn], acc.at[ida.at[c + A - NB, 0]],
                        sems_s[bn]).wait()
                pltpu.async_copy(
                    y_hbm.at[isa.at[pl.ds((c + A) * CH, CH)]],
                    rows[bn], sems_g[bn])
        return 0
    lax.fori_loop(0, NCHUNK // NB, outer, 0)

    for j in range(NB):
        c = NCHUNK - NB + j
        b = c % NB
        pltpu.make_async_copy(
            rows[b], acc.at[ida.at[c, 0]], sems_s[b]).wait()

    plsc.subcore_barrier()
    pltpu.sync_copy(acc.at[pl.ds(sid * RPT, RPT)],
                    z_out.at[cid, pl.ds(sid * RPT, RPT)])


@functools.lru_cache(maxsize=None)
def _sc_kernels():
    mesh = plsc.VectorSubcoreMesh(
        core_axis_name="c", subcore_axis_name="s",
        num_cores=NC, num_subcores=NS)
    prep = pl.kernel(
        _sc_prep_body,
        out_type=[
            jax.ShapeDtypeStruct((NC, NP, DE), jnp.float32),
            jax.ShapeDtypeStruct((NC, NP, DE), jnp.float32),
            jax.ShapeDtypeStruct((NC, NP, DE), jnp.float32),
        ],
        mesh=mesh,
        scratch_types=[
            pltpu.VMEM((CH,), jnp.int32),
            pltpu.VMEM((CH,), jnp.int32),
            pltpu.VMEM((CH, DE), jnp.float32),
            pltpu.VMEM((CH, DE), jnp.float32),
            pltpu.VMEM_SHARED((NP, DE), jnp.float32),
            pltpu.VMEM_SHARED((NP, DE), jnp.float32),
            pltpu.VMEM_SHARED((NP, DE), jnp.float32),
        ],
        compiler_params=pltpu.CompilerParams(use_tc_tiling_on_sc=False),
        name="sc_prep",
    )
    agg = pl.kernel(
        _sc_edge_agg_body,
        out_type=jax.ShapeDtypeStruct((NC, NP, H), jnp.float32),
        mesh=mesh,
        scratch_types=(
            [pltpu.VMEM((CH, H), jnp.float32) for _ in range(NB)]
            + [pltpu.SemaphoreType.DMA for _ in range(2 * NB)]
            + [pltpu.VMEM((EPT,), jnp.int32),
               pltpu.VMEM((NCHUNK, 1, CH), jnp.int32),
               pltpu.VMEM_SHARED((NP, H), jnp.float32)]
        ),
        compiler_params=pltpu.CompilerParams(use_tc_tiling_on_sc=False),
        name="sc_edge_agg",
    )
    return prep, agg


def _sc_prep(ea, src, dst):
    return _sc_kernels()[0](ea, src, dst)


def _sc_edge_agg(y, src, dst3):
    return _sc_kernels()[1](y, src, dst3)


def _t1_body(x_ref, se0_ref, se1_ref, cs0_ref, cs1_ref, cd0_ref, cd1_ref,
             wet_ref, be_ref, w1lt_ref, w1rt_ref,
             y1_ref, r1_ref, inv_ref):
    sege = se0_ref[...] + se1_ref[...]
    csrc = (cs0_ref[...] + cs1_ref[...])[:, 0:1]
    cdst = (cd0_ref[...] + cd1_ref[...])[:, 0:1]
    inv = 1.0 / jnp.maximum(cdst, 1.0)
    inv_ref[...] = jnp.broadcast_to(inv, (128, H))
    h0 = (x_ref[...]
          + jnp.dot(sege, wet_ref[...], precision=_HI)
          + csrc * be_ref[...])
    y1_ref[...] = jnp.dot(h0, w1lt_ref[...], precision=_HI)
    r1_ref[...] = jnp.dot(h0, w1rt_ref[...], precision=_HI)


def _combine_body(z0_ref, z1_ref, inv_ref, r_ref, b_ref, wlt_ref, wrt_ref,
                  y_ref, rn_ref):
    h = jax.nn.relu((z0_ref[...] + z1_ref[...]) * inv_ref[...]
                    + b_ref[...] + r_ref[...])
    y_ref[...] = jnp.dot(h, wlt_ref[...], precision=_HI)
    rn_ref[...] = jnp.dot(h, wrt_ref[...], precision=_HI)


def _final_body(z0_ref, z1_ref, inv_ref, r_ref, b_ref, batch_ref,
                wfct_ref, bfc_ref, out_ref, accp, accc):
    b = pl.program_id(0)
    h = jax.nn.relu((z0_ref[...] + z1_ref[...]) * inv_ref[...]
                    + b_ref[...] + r_ref[...])
    gid = jnp.broadcast_to(batch_ref[0], (G, 128))
    sel = jnp.where(gid == lax.broadcasted_iota(jnp.int32, (G, 128), 0),
                    1.0, 0.0)

    @pl.when(b == 0)
    def _():
        accp[...] = jnp.zeros((G, H), jnp.float32)
        accc[...] = jnp.zeros((G, H), jnp.float32)

    accp[...] += jnp.dot(sel, h, precision=_HI)
    accc[...] += jnp.dot(sel, jnp.ones((128, H), jnp.float32), precision=_HI)

    @pl.when(b == pl.num_programs(0) - 1)
    def _():
        pooled = accp[...] / jnp.maximum(accc[...], 1.0)
        out_ref[...] = jnp.dot(pooled, wfct_ref[...], precision=_HI) + bfc_ref[...]


def _row_spec(w):
    return pl.BlockSpec((128, w), lambda b: (b, 0))


def _full_spec(r, w):
    return pl.BlockSpec((r, w), lambda b: (0, 0))


def _tc_prep(x_pad, se0, se1, cs0, cs1, cd0, cd1, wet, be_row, w1lt, w1rt):
    grid = NP // 128
    return pl.pallas_call(
        _t1_body,
        grid=(grid,),
        in_specs=[
            _row_spec(D), _row_spec(DE), _row_spec(DE),
            _row_spec(DE), _row_spec(DE), _row_spec(DE), _row_spec(DE),
            _full_spec(DE, D), _full_spec(1, D),
            _full_spec(D, H), _full_spec(D, H),
        ],
        out_specs=[_row_spec(H), _row_spec(H), _row_spec(H)],
        out_shape=[
            jax.ShapeDtypeStruct((NP, H), jnp.float32),
            jax.ShapeDtypeStruct((NP, H), jnp.float32),
            jax.ShapeDtypeStruct((NP, H), jnp.float32),
        ],
    )(x_pad, se0, se1, cs0, cs1, cd0, cd1, wet, be_row, w1lt, w1rt)


def _tc_combine(z0, z1, inv, r, b_row, wlt, wrt):
    grid = NP // 128
    return pl.pallas_call(
        _combine_body,
        grid=(grid,),
        in_specs=[
            _row_spec(H), _row_spec(H), _row_spec(H), _row_spec(H),
            _full_spec(1, H), _full_spec(H, H), _full_spec(H, H),
        ],
        out_specs=[_row_spec(H), _row_spec(H)],
        out_shape=[
            jax.ShapeDtypeStruct((NP, H), jnp.float32),
            jax.ShapeDtypeStruct((NP, H), jnp.float32),
        ],
    )(z0, z1, inv, r, b_row, wlt, wrt)


def _tc_final(z0, z1, inv, r, b_row, batch2d, wfct_pad, bfc_row):
    grid = NP // 128
    return pl.pallas_call(
        _final_body,
        grid=(grid,),
        in_specs=[
            _row_spec(H), _row_spec(H), _row_spec(H), _row_spec(H),
            _full_spec(1, H),
            pl.BlockSpec((1, 1, 128), lambda b: (b, 0, 0)),
            _full_spec(H, 128), _full_spec(1, 128),
        ],
        out_specs=pl.BlockSpec((G, 128), lambda b: (0, 0)),
        out_shape=jax.ShapeDtypeStruct((G, 128), jnp.float32),
        scratch_shapes=[
            pltpu.VMEM((G, H), jnp.float32),
            pltpu.VMEM((G, H), jnp.float32),
        ],
    )(z0, z1, inv, r, b_row, batch2d, wfct_pad, bfc_row)


@jax.jit
def kernel(x, edge_index, edge_attr, batch, W_edge, b_edge, W1l, b1l, W1r,
           W2l, b2l, W2r, W3l, b3l, W3r, Wfc, bfc):
    f32 = jnp.float32
    x_pad = jnp.pad(x, ((0, NP - N), (0, 0)))
    pad_e = EP - E
    src = jnp.concatenate([edge_index[0], jnp.full((pad_e,), N, jnp.int32)])
    dst = jnp.concatenate([edge_index[1], jnp.full((pad_e,), N, jnp.int32)])
    dst3 = dst.reshape(EP // CH, 1, CH)
    ea = jnp.pad(edge_attr, ((0, pad_e), (0, 0)))
    batch2d = jnp.pad(batch, (0, NP - N), constant_values=G).reshape(
        NP // 128, 1, 128)

    wet = W_edge.T
    be_row = b_edge.reshape(1, D)
    w1lt, w1rt = W1l.T, W1r.T
    w2lt, w2rt = W2l.T, W2r.T
    w3lt, w3rt = W3l.T, W3r.T
    b1 = b1l.reshape(1, H)
    b2 = b2l.reshape(1, H)
    b3 = b3l.reshape(1, H)
    wfct_pad = jnp.pad(Wfc.T, ((0, 0), (0, 128 - C)))
    bfc_row = jnp.pad(bfc, (0, 128 - C)).reshape(1, 128)

    sege, csrc, cdst = _sc_prep(ea, src, dst)

    y1, r1, inv = _tc_prep(
        x_pad, sege[0], sege[1], csrc[0], csrc[1], cdst[0], cdst[1],
        wet, be_row, w1lt, w1rt)

    z1 = _sc_edge_agg(y1, src, dst3)
    y2, r2 = _tc_combine(z1[0], z1[1], inv, r1, b1, w2lt, w2rt)
    z2 = _sc_edge_agg(y2, src, dst3)
    y3, r3 = _tc_combine(z2[0], z2[1], inv, r2, b2, w3lt, w3rt)
    z3 = _sc_edge_agg(y3, src, dst3)
    out = _tc_final(z3[0], z3[1], inv, r3, b3, batch2d, wfct_pad, bfc_row)
    return out[:, :C].astype(f32)


# speedup vs baseline: 1.0893x; 1.0893x over previous
import functools

import jax
import jax.numpy as jnp
from jax import lax
from jax.experimental import pallas as pl
from jax.experimental.pallas import tpu as pltpu
from jax.experimental.pallas import tpu_sc as plsc

N = 10000
E = 320000
D = 128
DE = 16
H = 64
G = 64
C = 2

NC = 2
NS = 16
NW = NC * NS

NP = 10240
EP = 327680
EPT = EP // NW
CH = 128
NCHUNK = EPT // CH
RPT = NP // NS

_HI = jax.lax.Precision.HIGHEST


def _sc_prep_body(ea_hbm, src3_hbm, dst3_hbm, sege_out, csrc_out, cdst_out,
                  *bufs):
    rows = bufs[0:NB]
    sems_e = bufs[NB:2 * NB]
    sems_x = bufs[2 * NB:5 * NB]
    ones_v = bufs[5 * NB]
    isa = bufs[5 * NB + 1]
    ida = bufs[5 * NB + 2]
    acc_e, acc_s, acc_d = bufs[5 * NB + 3:5 * NB + 6]
    cid = lax.axis_index("c")
    sid = lax.axis_index("s")
    wid = sid * NC + cid

    pltpu.sync_copy(src3_hbm.at[pl.ds(wid * NCHUNK, NCHUNK)], isa)
    pltpu.sync_copy(dst3_hbm.at[pl.ds(wid * NCHUNK, NCHUNK)], ida)

    def zfill(i, _):
        rows[0][i, :] = jnp.zeros((DE,), jnp.float32)
        return 0
    lax.fori_loop(0, CH, zfill, 0)
    for k in range(RPT // CH):
        off = sid * RPT + k * CH
        pltpu.sync_copy(rows[0], acc_e.at[pl.ds(off, CH)])
        pltpu.sync_copy(rows[0], acc_s.at[pl.ds(off, CH)])
        pltpu.sync_copy(rows[0], acc_d.at[pl.ds(off, CH)])

    def fill(i, _):
        ones_v[i, :] = jnp.ones((DE,), jnp.float32)
        return 0
    lax.fori_loop(0, CH, fill, 0)
    plsc.subcore_barrier()

    for b in range(A):
        pltpu.async_copy(ea_hbm.at[pl.ds((wid * NCHUNK + b) * CH, CH)],
                         rows[b], sems_e[b])

    def outer(i, _):
        for b in range(NB):
            c = i * NB + b
            base = wid * EPT + c * CH
            se = sems_x[3 * b]
            pltpu.make_async_copy(
                ea_hbm.at[pl.ds(base, CH)], rows[b], sems_e[b]).wait()
            pltpu.async_copy(rows[b], acc_e.at[isa.at[c, 0]], se, add=True)
            pltpu.sync_copy(ones_v, acc_s.at[isa.at[c, 0]], add=True)
            pltpu.sync_copy(ones_v, acc_d.at[ida.at[c, 0]], add=True)

            bn = (b + A) % NB

            @pl.when(c + A < NCHUNK)
            def _():
                @pl.when(c + A >= NB)
                def _():
                    pltpu.make_async_copy(
                        rows[bn], acc_e.at[isa.at[c + A - NB, 0]],
                        sems_x[3 * bn]).wait()
                pltpu.async_copy(
                    ea_hbm.at[pl.ds(wid * EPT + (c + A) * CH, CH)],
                    rows[bn], sems_e[bn])
        return 0
    lax.fori_loop(0, NCHUNK // NB, outer, 0)

    for j in range(NB):
        c = NCHUNK - NB + j
        b = c % NB
        pltpu.make_async_copy(
            rows[b], acc_e.at[isa.at[c, 0]], sems_x[3 * b]).wait()

    plsc.subcore_barrier()
    pltpu.sync_copy(acc_e.at[pl.ds(sid * RPT, RPT)],
                    sege_out.at[cid, pl.ds(sid * RPT, RPT)])
    pltpu.sync_copy(acc_s.at[pl.ds(sid * RPT, RPT)],
                    csrc_out.at[cid, pl.ds(sid * RPT, RPT)])
    pltpu.sync_copy(acc_d.at[pl.ds(sid * RPT, RPT)],
                    cdst_out.at[cid, pl.ds(sid * RPT, RPT)])


NB = 4
A = 2


def _sc_edge_agg_body(y_hbm, src_hbm, dst3_hbm, z_out, *bufs):
    rows = bufs[0:NB]
    sems_g = bufs[NB:2 * NB]
    sems_s = bufs[2 * NB:3 * NB]
    isa = bufs[3 * NB]
    ida = bufs[3 * NB + 1]
    acc = bufs[3 * NB + 2]
    cid = lax.axis_index("c")
    sid = lax.axis_index("s")
    wid = sid * NC + cid

    pltpu.sync_copy(src_hbm.at[pl.ds(wid * EPT, EPT)], isa)
    pltpu.sync_copy(dst3_hbm.at[pl.ds(wid * NCHUNK, NCHUNK)], ida)

    def zfill(i, _):
        for j in range(H // 16):
            rows[0][i, pl.ds(j * 16, 16)] = jnp.zeros((16,), jnp.float32)
        return 0
    lax.fori_loop(0, CH, zfill, 0)
    for k in range(RPT // CH):
        pltpu.sync_copy(rows[0], acc.at[pl.ds(sid * RPT + k * CH, CH)])
    plsc.subcore_barrier()

    for b in range(A):
        pltpu.async_copy(
            y_hbm.at[isa.at[pl.ds(b * CH, CH)]], rows[b], sems_g[b])

    def outer(i, _):
        for b in range(NB):
            c = i * NB + b
            pltpu.make_async_copy(
                y_hbm.at[isa.at[pl.ds(c * CH, CH)]],
                rows[b], sems_g[b]).wait()
            pltpu.async_copy(
                rows[b], acc.at[ida.at[c, 0]], sems_s[b], add=True)

            bn = (b + A) % NB

            @pl.when(c + A < NCHUNK)
            def _():
                @pl.when(c + A >= NB)
                def _():
                    pltpu.make_async_copy(
                        rows[bn], acc.at[ida.at[c + A - NB, 0]],
                        sems_s[bn]).wait()
                pltpu.async_copy(
                    y_hbm.at[isa.at[pl.ds((c + A) * CH, CH)]],
                    rows[bn], sems_g[bn])
        return 0
    lax.fori_loop(0, NCHUNK // NB, outer, 0)

    for j in range(NB):
        c = NCHUNK - NB + j
        b = c % NB
        pltpu.make_async_copy(
            rows[b], acc.at[ida.at[c, 0]], sems_s[b]).wait()

    plsc.subcore_barrier()
    pltpu.sync_copy(acc.at[pl.ds(sid * RPT, RPT)],
                    z_out.at[cid, pl.ds(sid * RPT, RPT)])


@functools.lru_cache(maxsize=None)
def _sc_kernels():
    mesh = plsc.VectorSubcoreMesh(
        core_axis_name="c", subcore_axis_name="s",
        num_cores=NC, num_subcores=NS)
    prep = pl.kernel(
        _sc_prep_body,
        out_type=[
            jax.ShapeDtypeStruct((NC, NP, DE), jnp.float32),
            jax.ShapeDtypeStruct((NC, NP, DE), jnp.float32),
            jax.ShapeDtypeStruct((NC, NP, DE), jnp.float32),
        ],
        mesh=mesh,
        scratch_types=(
            [pltpu.VMEM((CH, DE), jnp.float32) for _ in range(NB)]
            + [pltpu.SemaphoreType.DMA for _ in range(NB)]
            + [pltpu.SemaphoreType.DMA for _ in range(3 * NB)]
            + [pltpu.VMEM((CH, DE), jnp.float32),
               pltpu.VMEM((NCHUNK, 1, CH), jnp.int32),
               pltpu.VMEM((NCHUNK, 1, CH), jnp.int32),
               pltpu.VMEM_SHARED((NP, DE), jnp.float32),
               pltpu.VMEM_SHARED((NP, DE), jnp.float32),
               pltpu.VMEM_SHARED((NP, DE), jnp.float32)]
        ),
        compiler_params=pltpu.CompilerParams(use_tc_tiling_on_sc=False),
        name="sc_prep",
    )
    agg = pl.kernel(
        _sc_edge_agg_body,
        out_type=jax.ShapeDtypeStruct((NC, NP, H), jnp.float32),
        mesh=mesh,
        scratch_types=(
            [pltpu.VMEM((CH, H), jnp.float32) for _ in range(NB)]
            + [pltpu.SemaphoreType.DMA for _ in range(2 * NB)]
            + [pltpu.VMEM((EPT,), jnp.int32),
               pltpu.VMEM((NCHUNK, 1, CH), jnp.int32),
               pltpu.VMEM_SHARED((NP, H), jnp.float32)]
        ),
        compiler_params=pltpu.CompilerParams(use_tc_tiling_on_sc=False),
        name="sc_edge_agg",
    )
    return prep, agg


def _sc_prep(ea, src3, dst3):
    return _sc_kernels()[0](ea, src3, dst3)


def _sc_edge_agg(y, src, dst3):
    return _sc_kernels()[1](y, src, dst3)


def _t1_body(x_ref, se0_ref, se1_ref, cs0_ref, cs1_ref, cd0_ref, cd1_ref,
             wet_ref, be_ref, w1lt_ref, w1rt_ref,
             y1_ref, r1_ref, inv_ref):
    sege = se0_ref[...] + se1_ref[...]
    csrc = (cs0_ref[...] + cs1_ref[...])[:, 0:1]
    cdst = (cd0_ref[...] + cd1_ref[...])[:, 0:1]
    inv = 1.0 / jnp.maximum(cdst, 1.0)
    inv_ref[...] = jnp.broadcast_to(inv, (128, H))
    h0 = (x_ref[...]
          + jnp.dot(sege, wet_ref[...], precision=_HI)
          + csrc * be_ref[...])
    y1_ref[...] = jnp.dot(h0, w1lt_ref[...], precision=_HI)
    r1_ref[...] = jnp.dot(h0, w1rt_ref[...], precision=_HI)


def _combine_body(z0_ref, z1_ref, inv_ref, r_ref, b_ref, wlt_ref, wrt_ref,
                  y_ref, rn_ref):
    h = jax.nn.relu((z0_ref[...] + z1_ref[...]) * inv_ref[...]
                    + b_ref[...] + r_ref[...])
    y_ref[...] = jnp.dot(h, wlt_ref[...], precision=_HI)
    rn_ref[...] = jnp.dot(h, wrt_ref[...], precision=_HI)


def _final_body(z0_ref, z1_ref, inv_ref, r_ref, b_ref, batch_ref,
                wfct_ref, bfc_ref, out_ref, accp, accc):
    b = pl.program_id(0)
    h = jax.nn.relu((z0_ref[...] + z1_ref[...]) * inv_ref[...]
                    + b_ref[...] + r_ref[...])
    gid = jnp.broadcast_to(batch_ref[0], (G, 128))
    sel = jnp.where(gid == lax.broadcasted_iota(jnp.int32, (G, 128), 0),
                    1.0, 0.0)

    @pl.when(b == 0)
    def _():
        accp[...] = jnp.zeros((G, H), jnp.float32)
        accc[...] = jnp.zeros((G, H), jnp.float32)

    accp[...] += jnp.dot(sel, h, precision=_HI)
    accc[...] += jnp.dot(sel, jnp.ones((128, H), jnp.float32), precision=_HI)

    @pl.when(b == pl.num_programs(0) - 1)
    def _():
        pooled = accp[...] / jnp.maximum(accc[...], 1.0)
        out_ref[...] = jnp.dot(pooled, wfct_ref[...], precision=_HI) + bfc_ref[...]


def _row_spec(w):
    return pl.BlockSpec((128, w), lambda b: (b, 0))


def _full_spec(r, w):
    return pl.BlockSpec((r, w), lambda b: (0, 0))


def _tc_prep(x_pad, se0, se1, cs0, cs1, cd0, cd1, wet, be_row, w1lt, w1rt):
    grid = NP // 128
    return pl.pallas_call(
        _t1_body,
        grid=(grid,),
        in_specs=[
            _row_spec(D), _row_spec(DE), _row_spec(DE),
            _row_spec(DE), _row_spec(DE), _row_spec(DE), _row_spec(DE),
            _full_spec(DE, D), _full_spec(1, D),
            _full_spec(D, H), _full_spec(D, H),
        ],
        out_specs=[_row_spec(H), _row_spec(H), _row_spec(H)],
        out_shape=[
            jax.ShapeDtypeStruct((NP, H), jnp.float32),
            jax.ShapeDtypeStruct((NP, H), jnp.float32),
            jax.ShapeDtypeStruct((NP, H), jnp.float32),
        ],
    )(x_pad, se0, se1, cs0, cs1, cd0, cd1, wet, be_row, w1lt, w1rt)


def _tc_combine(z0, z1, inv, r, b_row, wlt, wrt):
    grid = NP // 128
    return pl.pallas_call(
        _combine_body,
        grid=(grid,),
        in_specs=[
            _row_spec(H), _row_spec(H), _row_spec(H), _row_spec(H),
            _full_spec(1, H), _full_spec(H, H), _full_spec(H, H),
        ],
        out_specs=[_row_spec(H), _row_spec(H)],
        out_shape=[
            jax.ShapeDtypeStruct((NP, H), jnp.float32),
            jax.ShapeDtypeStruct((NP, H), jnp.float32),
        ],
    )(z0, z1, inv, r, b_row, wlt, wrt)


def _tc_final(z0, z1, inv, r, b_row, batch2d, wfct_pad, bfc_row):
    grid = NP // 128
    return pl.pallas_call(
        _final_body,
        grid=(grid,),
        in_specs=[
            _row_spec(H), _row_spec(H), _row_spec(H), _row_spec(H),
            _full_spec(1, H),
            pl.BlockSpec((1, 1, 128), lambda b: (b, 0, 0)),
            _full_spec(H, 128), _full_spec(1, 128),
        ],
        out_specs=pl.BlockSpec((G, 128), lambda b: (0, 0)),
        out_shape=jax.ShapeDtypeStruct((G, 128), jnp.float32),
        scratch_shapes=[
            pltpu.VMEM((G, H), jnp.float32),
            pltpu.VMEM((G, H), jnp.float32),
        ],
    )(z0, z1, inv, r, b_row, batch2d, wfct_pad, bfc_row)


@jax.jit
def kernel(x, edge_index, edge_attr, batch, W_edge, b_edge, W1l, b1l, W1r,
           W2l, b2l, W2r, W3l, b3l, W3r, Wfc, bfc):
    f32 = jnp.float32
    x_pad = jnp.pad(x, ((0, NP - N), (0, 0)))
    pad_e = EP - E
    src = jnp.concatenate([edge_index[0], jnp.full((pad_e,), N, jnp.int32)])
    dst = jnp.concatenate([edge_index[1], jnp.full((pad_e,), N, jnp.int32)])
    src3 = src.reshape(EP // CH, 1, CH)
    dst3 = dst.reshape(EP // CH, 1, CH)
    ea = jnp.pad(edge_attr, ((0, pad_e), (0, 0)))
    batch2d = jnp.pad(batch, (0, NP - N), constant_values=G).reshape(
        NP // 128, 1, 128)

    wet = W_edge.T
    be_row = b_edge.reshape(1, D)
    w1lt, w1rt = W1l.T, W1r.T
    w2lt, w2rt = W2l.T, W2r.T
    w3lt, w3rt = W3l.T, W3r.T
    b1 = b1l.reshape(1, H)
    b2 = b2l.reshape(1, H)
    b3 = b3l.reshape(1, H)
    wfct_pad = jnp.pad(Wfc.T, ((0, 0), (0, 128 - C)))
    bfc_row = jnp.pad(bfc, (0, 128 - C)).reshape(1, 128)

    sege, csrc, cdst = _sc_prep(ea, src3, dst3)

    y1, r1, inv = _tc_prep(
        x_pad, sege[0], sege[1], csrc[0], csrc[1], cdst[0], cdst[1],
        wet, be_row, w1lt, w1rt)

    z1 = _sc_edge_agg(y1, src, dst3)
    y2, r2 = _tc_combine(z1[0], z1[1], inv, r1, b1, w2lt, w2rt)
    z2 = _sc_edge_agg(y2, src, dst3)
    y3, r3 = _tc_combine(z2[0], z2[1], inv, r2, b2, w3lt, w3rt)
    z3 = _sc_edge_agg(y3, src, dst3)
    out = _tc_final(z3[0], z3[1], inv, r3, b3, batch2d, wfct_pad, bfc_row)
    return out[:, :C].astype(f32)
